# baseline (speedup 1.0000x reference)
import functools

import jax
import jax.numpy as jnp
from jax import lax
from jax.experimental import pallas as pl
from jax.experimental.pallas import tpu as pltpu
from jax.experimental.pallas import tpu_sc as plsc

BATCH = 4
SEQ = 8192
DIM = 64
TOTAL = BATCH * SEQ

NUM_CORES = 2
NUM_SUBCORES = 16
NUM_WORKERS = NUM_CORES * NUM_SUBCORES
PER_WORKER = TOTAL // NUM_WORKERS
CHUNK = 128
CHUNKS_PER_WORKER = PER_WORKER // CHUNK


def _rope_gather_kernel(pos_hbm, cos_hbm, sin_hbm, cos_out, sin_out,
                        idx_v, buf_cos, buf_sin, sem_c, sem_s):
    wid = lax.axis_index("s") * NUM_CORES + lax.axis_index("c")
    row0 = wid * CHUNKS_PER_WORKER
    base = wid * PER_WORKER

    pltpu.sync_copy(pos_hbm.at[pl.ds(row0, CHUNKS_PER_WORKER)], idx_v)

    for j in range(CHUNKS_PER_WORKER):
        idx_row = idx_v.at[j]
        cp_c = pltpu.async_copy(cos_hbm.at[idx_row], buf_cos, sem_c)
        cp_s = pltpu.async_copy(sin_hbm.at[idx_row], buf_sin, sem_s)
        cp_c.wait()
        pltpu.sync_copy(buf_cos, cos_out.at[pl.ds(base + j * CHUNK, CHUNK)])
        cp_s.wait()
        pltpu.sync_copy(buf_sin, sin_out.at[pl.ds(base + j * CHUNK, CHUNK)])


@jax.jit
def _rope_gather(pos_flat, cos_cache, sin_cache):
    mesh = plsc.VectorSubcoreMesh(core_axis_name="c", subcore_axis_name="s")
    out_t = jax.ShapeDtypeStruct((TOTAL, DIM), jnp.float32)
    scratch = [
        pltpu.VMEM((CHUNKS_PER_WORKER, CHUNK), jnp.int32),
        pltpu.VMEM((CHUNK, DIM), jnp.float32),
        pltpu.VMEM((CHUNK, DIM), jnp.float32),
        pltpu.SemaphoreType.DMA,
        pltpu.SemaphoreType.DMA,
    ]
    return pl.kernel(
        _rope_gather_kernel,
        out_type=(out_t, out_t),
        mesh=mesh,
        scratch_types=scratch,
        compiler_params=pltpu.CompilerParams(use_tc_tiling_on_sc=False),
    )(pos_flat, cos_cache, sin_cache)


def kernel(position_ids, cos_cache, sin_cache):
    pos_flat = position_ids.astype(jnp.int32).reshape(TOTAL // CHUNK, CHUNK)
    cos, sin = _rope_gather(pos_flat, cos_cache, sin_cache)
    shape = (*position_ids.shape, DIM)
    return cos.reshape(shape), sin.reshape(shape)


# speedup vs baseline: 1.2008x; 1.2008x over previous
import jax
import jax.numpy as jnp
from jax import lax
from jax.experimental import pallas as pl
from jax.experimental.pallas import tpu as pltpu
from jax.experimental.pallas import tpu_sc as plsc

BATCH = 4
SEQ = 8192
DIM = 64
TOTAL = BATCH * SEQ

NUM_CORES = 2
NUM_SUBCORES = 16
NUM_WORKERS = NUM_CORES * NUM_SUBCORES
PER_WORKER = TOTAL // NUM_WORKERS
CHUNK = 128
NCHUNK = PER_WORKER // CHUNK


def _split_rows(gbuf, cbuf, sbuf):
    def row(r, carry):
        for c in range(DIM // 16):
            cbuf[r, pl.ds(c * 16, 16)] = gbuf[r, pl.ds(c * 16, 16)]
            sbuf[r, pl.ds(c * 16, 16)] = gbuf[r, pl.ds(DIM + c * 16, 16)]
        return carry
    lax.fori_loop(0, CHUNK, row, 0)


def _rope_kernel(pos_hbm, tab_hbm, cos_out, sin_out,
                 idx_v, g0, g1, c0, c1, s0, s1,
                 sem_g0, sem_g1, sem_c0, sem_c1, sem_s0, sem_s1):
    wid = lax.axis_index("s") * NUM_CORES + lax.axis_index("c")
    row0 = wid * NCHUNK
    base = wid * PER_WORKER

    gbuf = (g0, g1)
    cbuf = (c0, c1)
    sbuf = (s0, s1)
    sem_g = (sem_g0, sem_g1)
    sem_c = (sem_c0, sem_c1)
    sem_s = (sem_s0, sem_s1)

    pltpu.sync_copy(pos_hbm.at[pl.ds(row0, NCHUNK)], idx_v)

    gcp = [None] * NCHUNK
    wcp_c = [None, None]
    wcp_s = [None, None]
    gcp[0] = pltpu.async_copy(tab_hbm.at[idx_v.at[0]], gbuf[0], sem_g[0])
    for j in range(NCHUNK):
        s = j % 2
        if j + 1 < NCHUNK:
            gcp[j + 1] = pltpu.async_copy(
                tab_hbm.at[idx_v.at[j + 1]], gbuf[(j + 1) % 2], sem_g[(j + 1) % 2])
        gcp[j].wait()
        if j >= 2:
            wcp_c[s].wait()
            wcp_s[s].wait()
        _split_rows(gbuf[s], cbuf[s], sbuf[s])
        dst = pl.ds(base + j * CHUNK, CHUNK)
        wcp_c[s] = pltpu.async_copy(cbuf[s], cos_out.at[dst], sem_c[s])
        wcp_s[s] = pltpu.async_copy(sbuf[s], sin_out.at[dst], sem_s[s])
    for s in range(2):
        wcp_c[s].wait()
        wcp_s[s].wait()


@jax.jit
def _rope_gather(pos2d, table):
    mesh = plsc.VectorSubcoreMesh(core_axis_name="c", subcore_axis_name="s")
    out_t = jax.ShapeDtypeStruct((TOTAL, DIM), jnp.float32)
    scratch = [
        pltpu.VMEM((NCHUNK, CHUNK), jnp.int32),
        pltpu.VMEM((CHUNK, 2 * DIM), jnp.float32),
        pltpu.VMEM((CHUNK, 2 * DIM), jnp.float32),
        pltpu.VMEM((CHUNK, DIM), jnp.float32),
        pltpu.VMEM((CHUNK, DIM), jnp.float32),
        pltpu.VMEM((CHUNK, DIM), jnp.float32),
        pltpu.VMEM((CHUNK, DIM), jnp.float32),
        pltpu.SemaphoreType.DMA,
        pltpu.SemaphoreType.DMA,
        pltpu.SemaphoreType.DMA,
        pltpu.SemaphoreType.DMA,
        pltpu.SemaphoreType.DMA,
        pltpu.SemaphoreType.DMA,
    ]
    return pl.kernel(
        _rope_kernel,
        out_type=(out_t, out_t),
        mesh=mesh,
        scratch_types=scratch,
    )(pos2d, table)


def kernel(position_ids, cos_cache, sin_cache):
    pos2d = position_ids.astype(jnp.int32).reshape(TOTAL // CHUNK, CHUNK)
    table = jnp.concatenate([cos_cache, sin_cache], axis=1)
    cos, sin = _rope_gather(pos2d, table)
    shape = (*position_ids.shape, DIM)
    return cos.reshape(shape), sin.reshape(shape)
